# Initial kernel scaffold; baseline (speedup 1.0000x reference)
import jax
import jax.numpy as jnp
from jax.experimental import pallas as pl


def kernel(injected_radiance, kernel_val, detection_weight, kernel_row, kernel_col, delay):
    raise NotImplementedError("write your pallas kernel here")


# speedup vs baseline: 2.9935x; 2.9935x over previous
import functools

import jax
import jax.numpy as jnp
from jax import lax
from jax.experimental import pallas as pl
from jax.experimental.pallas import tpu as pltpu
from jax.experimental.pallas import tpu_sc as plsc

R = 65536
T = 96
NNZ = 524288
NC = 2
NS = 16
NW = NC * NS
L = 16

NCHUNK = 8
CHUNK_ROWS = R // NCHUNK
CHUNK_SHIFT = 13
ROWS_PER_SUB = CHUNK_ROWS // NS
WB_BLK = 128

EDGES_PER_SUB = NNZ // NS
SEG = 4096
NSEG = EDGES_PER_SUB // SEG
GB = 128
NB_MAX = SEG // GB

E0_ROWS = R // NW


def _iota16():
    return lax.iota(jnp.int32, L)


def _splat(ref, i):
    return plsc.load_gather(ref, [jnp.full((L,), i, jnp.int32)])


def _bounce_body(with_echo0, e_old, kval, w, krow, kcol, kdel,
                 e_new, echo_out,
                 acc_sh, eb_row, eb_col, eb_val, eb_del,
                 cb_rl, cb_col, cb_val, cb_del,
                 idxbuf, colbuf, rowbuf, stage, wb, wvec, eacc):
    c = lax.axis_index("c")
    s = lax.axis_index("s")
    wid = c * NS + s
    iota = _iota16()
    zf = jnp.zeros((L,), jnp.float32)
    zi = jnp.zeros((L,), jnp.int32)

    for t6 in range(6):
        eacc[pl.ds(t6 * L, L)] = zf

    def detect_rows(block_ref, w_off, nrows):
        def body(r, carry):
            wv = _splat(wvec, w_off + r)
            ri = jnp.full((L,), r, jnp.int32)
            return tuple(
                carry[t] + wv * plsc.load_gather(block_ref, [ri, iota + t * L])
                for t in range(6)
            )
        acc = lax.fori_loop(0, nrows, body,
                            tuple(zf for _ in range(6)))
        for t6 in range(6):
            eacc[pl.ds(t6 * L, L)] = eacc[pl.ds(t6 * L, L)] + acc[t6]

    if with_echo0:
        base = wid * E0_ROWS
        pltpu.sync_copy(w.at[pl.ds(base, E0_ROWS)], wvec)

        def e0blk(blk, _):
            pltpu.sync_copy(e_old.at[pl.ds(base + blk * WB_BLK, WB_BLK)], wb)
            detect_rows(wb, blk * WB_BLK, WB_BLK)
            return 0
        lax.fori_loop(0, E0_ROWS // WB_BLK, e0blk, 0)

    my_row0 = s * ROWS_PER_SUB

    def do_pass(p, _):
        chunk = c * (NCHUNK // NC) + p
        chunk_base = chunk * CHUNK_ROWS

        def zrow(r, _):
            ri = jnp.full((L,), r, jnp.int32)
            for t6 in range(6):
                plsc.store_scatter(wb, [ri, iota + t6 * L], zf)
            return 0
        lax.fori_loop(0, WB_BLK, zrow, 0)

        def zblk(blk, _):
            pltpu.sync_copy(wb, acc_sh.at[pl.ds(my_row0 + blk * WB_BLK, WB_BLK)])
            return 0
        lax.fori_loop(0, ROWS_PER_SUB // WB_BLK, zblk, 0)
        plsc.subcore_barrier()

        def do_seg(seg, _):
            seg_base = s * EDGES_PER_SUB + seg * SEG
            pltpu.sync_copy(krow.at[pl.ds(seg_base, SEG)], eb_row)
            pltpu.sync_copy(kcol.at[pl.ds(seg_base, SEG)], eb_col)
            pltpu.sync_copy(kval.at[pl.ds(seg_base, SEG)], eb_val)
            pltpu.sync_copy(kdel.at[pl.ds(seg_base, SEG)], eb_del)

            def compress(b, cnt):
                off = b * L
                r16 = eb_row[pl.ds(off, L)]
                m = lax.shift_right_logical(r16, CHUNK_SHIFT) == chunk
                plsc.store_compressed(cb_rl.at[pl.ds(cnt, L)],
                                      r16 & (CHUNK_ROWS - 1), mask=m)
                plsc.store_compressed(cb_col.at[pl.ds(cnt, L)],
                                      eb_col[pl.ds(off, L)], mask=m)
                plsc.store_compressed(cb_val.at[pl.ds(cnt, L)],
                                      eb_val[pl.ds(off, L)], mask=m)
                plsc.store_compressed(cb_del.at[pl.ds(cnt, L)],
                                      eb_del[pl.ds(off, L)], mask=m)
                return cnt + jnp.max(plsc.all_reduce_population_count(m))
            cnt = lax.fori_loop(0, SEG // L, compress, jnp.int32(0))

            for i in range(GB // L):
                cb_rl[pl.ds(cnt + i * L, L)] = zi
                cb_col[pl.ds(cnt + i * L, L)] = zi
                cb_val[pl.ds(cnt + i * L, L)] = zf
                cb_del[pl.ds(cnt + i * L, L)] = zi

            nb = (cnt + GB - 1) // GB

            def batch(j, _):
                for i in range(GB // L):
                    idxbuf[pl.ds(i * L, L)] = cb_rl[pl.ds(j * GB + i * L, L)]
                    colbuf[pl.ds(i * L, L)] = cb_col[pl.ds(j * GB + i * L, L)]
                pltpu.sync_copy(e_old.at[colbuf], rowbuf)

                def edge(e, _):
                    ge = j * GB + e
                    dv = _splat(cb_del, ge)
                    vv = _splat(cb_val, ge)
                    ei = jnp.full((L,), e, jnp.int32)
                    for t6 in range(6):
                        tv = iota + t6 * L
                        st = tv - dv
                        m = st >= 0
                        g = plsc.load_gather(rowbuf, [ei, st], mask=m)
                        plsc.store_scatter(stage, [ei, tv],
                                           jnp.where(m, g, 0.0) * vv)
                    return 0
                lax.fori_loop(0, GB, edge, 0)
                pltpu.sync_copy(stage, acc_sh.at[idxbuf], add=True)
                return 0
            lax.fori_loop(0, nb, batch, 0)
            return 0
        lax.fori_loop(0, NSEG, do_seg, 0)

        plsc.subcore_barrier()

        pltpu.sync_copy(w.at[pl.ds(chunk_base + my_row0, ROWS_PER_SUB)],
                        wvec.at[pl.ds(0, ROWS_PER_SUB)])

        def wblk(blk, _):
            r0 = my_row0 + blk * WB_BLK
            pltpu.sync_copy(acc_sh.at[pl.ds(r0, WB_BLK)], wb)
            detect_rows(wb, blk * WB_BLK, WB_BLK)
            pltpu.sync_copy(wb, e_new.at[pl.ds(chunk_base + r0, WB_BLK)])
            return 0
        lax.fori_loop(0, ROWS_PER_SUB // WB_BLK, wblk, 0)
        return 0

    lax.fori_loop(0, NCHUNK // NC, do_pass, 0)
    pltpu.sync_copy(eacc, echo_out.at[wid])


@functools.lru_cache(maxsize=None)
def _bounce(with_echo0: bool):
    mesh = plsc.VectorSubcoreMesh(core_axis_name="c", subcore_axis_name="s")
    return pl.kernel(
        functools.partial(_bounce_body, with_echo0),
        out_type=[
            jax.ShapeDtypeStruct((R, T), jnp.float32),
            jax.ShapeDtypeStruct((NW, T), jnp.float32),
        ],
        mesh=mesh,
        compiler_params=pltpu.CompilerParams(needs_layout_passes=False,
                                             use_tc_tiling_on_sc=False),
        scratch_types=[
            pltpu.VMEM_SHARED((CHUNK_ROWS, T), jnp.float32),
            pltpu.VMEM((SEG,), jnp.int32),
            pltpu.VMEM((SEG,), jnp.int32),
            pltpu.VMEM((SEG,), jnp.float32),
            pltpu.VMEM((SEG,), jnp.int32),
            pltpu.VMEM((SEG + GB,), jnp.int32),
            pltpu.VMEM((SEG + GB,), jnp.int32),
            pltpu.VMEM((SEG + GB,), jnp.float32),
            pltpu.VMEM((SEG + GB,), jnp.int32),
            pltpu.VMEM((GB,), jnp.int32),
            pltpu.VMEM((GB,), jnp.int32),
            pltpu.VMEM((GB, T), jnp.float32),
            pltpu.VMEM((GB, T), jnp.float32),
            pltpu.VMEM((WB_BLK, T), jnp.float32),
            pltpu.VMEM((E0_ROWS,), jnp.float32),
            pltpu.VMEM((T,), jnp.float32),
        ],
    )


def kernel(injected_radiance, kernel_val, detection_weight, kernel_row,
           kernel_col, delay):
    e1, echo_a = _bounce(True)(injected_radiance, kernel_val, detection_weight,
                               kernel_row, kernel_col, delay)
    _, echo_b = _bounce(False)(e1, kernel_val, detection_weight,
                               kernel_row, kernel_col, delay)
    return jnp.sum(echo_a, axis=0) + jnp.sum(echo_b, axis=0)
